# Initial kernel scaffold; baseline (speedup 1.0000x reference)
import jax
import jax.numpy as jnp
from jax.experimental import pallas as pl


def kernel(x, edge_attr, params, edge_index):
    raise NotImplementedError("write your pallas kernel here")


# speedup vs baseline: 1.6594x; 1.6594x over previous
import functools

import jax
import jax.numpy as jnp
from jax import lax
from jax.experimental import pallas as pl
from jax.experimental.pallas import tpu as pltpu
from jax.experimental.pallas import tpu_sc as plsc

NN = 10000
NE = 320000
HID = 128
NC = 2
NS = 16
NW = NC * NS
EPW = NE // NW
CH = 80
NCHUNK = EPW // CH
NNP = 10240
RPS = NNP // NS
F32 = jnp.float32


def _emb_h_body(x_ref, w_ref, b_ref, o_ref):
    o_ref[...] = jnp.dot(x_ref[...], w_ref[...],
                         preferred_element_type=F32) + b_ref[...]


def _emb_h(x, W, b):
    return pl.pallas_call(
        _emb_h_body,
        out_shape=jax.ShapeDtypeStruct((NN, HID), F32),
    )(x, W, b.reshape(1, HID))


def _proj_body(h_ref, w_ref, b_ref, o_ref):
    h = h_ref[...]
    for i in range(4):
        o_ref[i] = jnp.dot(h, w_ref[i], preferred_element_type=F32) + b_ref[i]


def _proj(h, Wst, bst):
    return pl.pallas_call(
        _proj_body,
        out_shape=jax.ShapeDtypeStruct((4, NN, HID), F32),
    )(h, Wst, bst)


def _edge_prep0_body(ea_ref, we_ref, be_ref, cw_ref, cb_ref, e_ref, ce_ref):
    e = ea_ref[...] * we_ref[...] + be_ref[...]
    e_ref[...] = e
    ce_ref[...] = jnp.dot(e, cw_ref[...], preferred_element_type=F32) + cb_ref[...]


def _edge_prep0(edge_attr, We, be, CW, Cb, blk=3200):
    grid = NE // blk
    return pl.pallas_call(
        _edge_prep0_body,
        grid=(grid,),
        in_specs=[
            pl.BlockSpec((blk, 1), lambda i: (i, 0)),
            pl.BlockSpec((1, HID), lambda i: (0, 0)),
            pl.BlockSpec((1, HID), lambda i: (0, 0)),
            pl.BlockSpec((HID, HID), lambda i: (0, 0)),
            pl.BlockSpec((1, HID), lambda i: (0, 0)),
        ],
        out_specs=[
            pl.BlockSpec((blk, HID), lambda i: (i, 0)),
            pl.BlockSpec((blk, HID), lambda i: (i, 0)),
        ],
        out_shape=[
            jax.ShapeDtypeStruct((NE, HID), F32),
            jax.ShapeDtypeStruct((NE, HID), F32),
        ],
    )(edge_attr, We.reshape(1, HID), be.reshape(1, HID), CW, Cb.reshape(1, HID))


def _edge_prep_body(ein_ref, enew_ref, st_ref, g_ref, bt_ref, cw_ref, cb_ref,
                    e_ref, ce_ref):
    st = st_ref[...]
    mean = jnp.sum(st[:, 0, :], axis=0, keepdims=True) * (1.0 / NE)
    meansq = jnp.sum(st[:, 1, :], axis=0, keepdims=True) * (1.0 / NE)
    var = meansq - mean * mean
    inv = g_ref[...] * lax.rsqrt(var + 1e-5)
    en = (enew_ref[...] - mean) * inv + bt_ref[...]
    e = ein_ref[...] + jnp.maximum(en, 0.0)
    e_ref[...] = e
    ce_ref[...] = jnp.dot(e, cw_ref[...], preferred_element_type=F32) + cb_ref[...]


def _edge_prep(e_in, e_new, estats, gamma, beta, CW, Cb, blk=3200):
    grid = NE // blk
    return pl.pallas_call(
        _edge_prep_body,
        grid=(grid,),
        in_specs=[
            pl.BlockSpec((blk, HID), lambda i: (i, 0)),
            pl.BlockSpec((blk, HID), lambda i: (i, 0)),
            pl.BlockSpec((NW, 2, HID), lambda i: (0, 0, 0)),
            pl.BlockSpec((1, HID), lambda i: (0, 0)),
            pl.BlockSpec((1, HID), lambda i: (0, 0)),
            pl.BlockSpec((HID, HID), lambda i: (0, 0)),
            pl.BlockSpec((1, HID), lambda i: (0, 0)),
        ],
        out_specs=[
            pl.BlockSpec((blk, HID), lambda i: (i, 0)),
            pl.BlockSpec((blk, HID), lambda i: (i, 0)),
        ],
        out_shape=[
            jax.ShapeDtypeStruct((NE, HID), F32),
            jax.ShapeDtypeStruct((NE, HID), F32),
        ],
    )(e_in, e_new, estats, gamma.reshape(1, HID), beta.reshape(1, HID),
      CW, Cb.reshape(1, HID))


def _node_update_body(h_ref, ah_ref, ms_ref, ss_ref, g_ref, bt_ref, o_ref):
    msum = ms_ref[0, :NN] + ms_ref[1, :NN]
    ssum = ss_ref[0, :NN] + ss_ref[1, :NN]
    t = ah_ref[...] + msum / (ssum + 1e-6)
    mu = jnp.mean(t, axis=0, keepdims=True)
    var = jnp.mean(t * t, axis=0, keepdims=True) - mu * mu
    hn = g_ref[...] * (t - mu) * lax.rsqrt(var + 1e-5) + bt_ref[...]
    o_ref[...] = h_ref[...] + jnp.maximum(hn, 0.0)


def _node_update(h, Ah, msum_p, ssum_p, gamma, beta):
    return pl.pallas_call(
        _node_update_body,
        out_shape=jax.ShapeDtypeStruct((NN, HID), F32),
    )(h, Ah, msum_p, ssum_p, gamma.reshape(1, HID), beta.reshape(1, HID))


def _mlp_body(h_ref, w1, b1, w2, b2, w3, b3, o_ref):
    y = jnp.maximum(jnp.dot(h_ref[...], w1[...], preferred_element_type=F32)
                    + b1[...], 0.0)
    y = jnp.maximum(jnp.dot(y, w2[...], preferred_element_type=F32)
                    + b2[...], 0.0)
    y = jnp.dot(y, w3[...], preferred_element_type=F32) + b3[...]
    n = jnp.sqrt(jnp.sum(y * y, axis=1, keepdims=True))
    o_ref[...] = y / jnp.maximum(n, 1e-12)


def _mlp(h, mlp_params):
    (w1, b1), (w2, b2), (w3, b3) = mlp_params
    return pl.pallas_call(
        _mlp_body,
        out_shape=jax.ShapeDtypeStruct((NN, w3.shape[1]), F32),
    )(h, w1, b1.reshape(1, -1), w2, b2.reshape(1, -1), w3, b3.reshape(1, -1))


def _sc_edge_body(bh, dh, eh, ce, srci, dsti,
                  enew, msum, ssum, estats,
                  idx_s, idx_d, bD, bE, bB, bC, stat, acc, sem):
    c = lax.axis_index("c")
    s = lax.axis_index("s")
    wid = s * NC + c
    ebase = wid * EPW
    rbase = s * RPS

    def zrow(r, _):
        for j in range(8):
            bC[r, pl.ds(j * 16, 16)] = jnp.zeros((16,), F32)
        return 0
    lax.fori_loop(0, CH, zrow, 0)
    for j in range(8):
        stat[0, pl.ds(j * 16, 16)] = jnp.zeros((16,), F32)
        stat[1, pl.ds(j * 16, 16)] = jnp.zeros((16,), F32)

    def zero_acc():
        def zcp(i, _):
            pltpu.sync_copy(bC, acc.at[pl.ds(rbase + i * CH, CH)])
            return 0
        lax.fori_loop(0, RPS // CH, zcp, 0)

    zero_acc()
    plsc.subcore_barrier()

    def chunk_a(t, _):
        base = ebase + t * CH
        pltpu.sync_copy(srci.at[pl.ds(base, CH)], idx_s)
        pltpu.sync_copy(dsti.at[pl.ds(base, CH)], idx_d)
        cp1 = pltpu.async_copy(dh.at[idx_s], bD, sem)
        cp2 = pltpu.async_copy(eh.at[idx_d], bE, sem)
        cp3 = pltpu.async_copy(bh.at[idx_s], bB, sem)
        cp4 = pltpu.async_copy(ce.at[pl.ds(base, CH)], bC, sem)
        cp1.wait(); cp2.wait(); cp3.wait(); cp4.wait()

        def row(r, _):
            for j in range(8):
                sl = pl.ds(j * 16, 16)
                en = bC[r, sl] + bD[r, sl] + bE[r, sl]
                bC[r, sl] = en
                stat[0, sl] = stat[0, sl] + en
                stat[1, sl] = stat[1, sl] + en * en
                sg = 1.0 / (1.0 + jnp.exp(-en))
                bB[r, sl] = sg * bB[r, sl]
            return 0
        lax.fori_loop(0, CH, row, 0)
        pltpu.sync_copy(bC, enew.at[pl.ds(base, CH)])
        pltpu.sync_copy(bB, acc.at[idx_d], add=True)
        return 0
    lax.fori_loop(0, NCHUNK, chunk_a, 0)
    pltpu.sync_copy(stat, estats.at[wid])
    plsc.subcore_barrier()
    pltpu.sync_copy(acc.at[pl.ds(rbase, RPS)], msum.at[c, pl.ds(rbase, RPS)])
    plsc.subcore_barrier()
    lax.fori_loop(0, CH, zrow, 0)
    zero_acc()
    plsc.subcore_barrier()

    def chunk_b(t, _):
        base = ebase + t * CH
        pltpu.sync_copy(dsti.at[pl.ds(base, CH)], idx_d)
        pltpu.sync_copy(enew.at[pl.ds(base, CH)], bC)

        def row(r, _):
            for j in range(8):
                sl = pl.ds(j * 16, 16)
                bB[r, sl] = 1.0 / (1.0 + jnp.exp(-bC[r, sl]))
            return 0
        lax.fori_loop(0, CH, row, 0)
        pltpu.sync_copy(bB, acc.at[idx_d], add=True)
        return 0
    lax.fori_loop(0, NCHUNK, chunk_b, 0)
    plsc.subcore_barrier()
    pltpu.sync_copy(acc.at[pl.ds(rbase, RPS)], ssum.at[c, pl.ds(rbase, RPS)])


_sc_edge = pl.kernel(
    _sc_edge_body,
    out_type=(
        jax.ShapeDtypeStruct((NE, HID), F32),
        jax.ShapeDtypeStruct((NC, NNP, HID), F32),
        jax.ShapeDtypeStruct((NC, NNP, HID), F32),
        jax.ShapeDtypeStruct((NW, 2, HID), F32),
    ),
    mesh=plsc.VectorSubcoreMesh(core_axis_name="c", subcore_axis_name="s"),
    scratch_types=[
        pltpu.VMEM((CH,), jnp.int32),
        pltpu.VMEM((CH,), jnp.int32),
        pltpu.VMEM((CH, HID), F32),
        pltpu.VMEM((CH, HID), F32),
        pltpu.VMEM((CH, HID), F32),
        pltpu.VMEM((CH, HID), F32),
        pltpu.VMEM((2, HID), F32),
        pltpu.MemorySpace.VMEM_SHARED((NNP, HID), F32),
        pltpu.SemaphoreType.DMA,
    ],
)


def kernel(x, edge_attr, params, edge_index):
    src = edge_index[0]
    dst = edge_index[1]
    layers = params['layers']

    h = _emb_h(x, *params['emb_h'])
    e = None
    e_new = None
    estats = None
    prev_bn = None
    for L in range(len(layers)):
        lay = layers[L]
        Wst = jnp.stack([lay[n][0] for n in ('A', 'B', 'D', 'E')])
        bst = jnp.stack([lay[n][1].reshape(1, HID) for n in ('A', 'B', 'D', 'E')])
        PJ = _proj(h, Wst, bst)
        Ah, Bh, Dh, Eh = PJ[0], PJ[1], PJ[2], PJ[3]
        CW, Cb = lay['C']
        if L == 0:
            We, be = params['emb_e']
            e, Ce = _edge_prep0(edge_attr, We, be, CW, Cb)
        else:
            e, Ce = _edge_prep(e, e_new, estats, prev_bn[0], prev_bn[1], CW, Cb)
        e_new, msum_p, ssum_p, estats = _sc_edge(Bh, Dh, Eh, Ce, src, dst)
        h = _node_update(h, Ah, msum_p, ssum_p, lay['bn_h'][0], lay['bn_h'][1])
        prev_bn = lay['bn_e']
    return _mlp(h, params['mlp'])
